# baseline (speedup 1.0000x reference)
import jax
import jax.numpy as jnp
from jax.experimental import pallas as pl
from jax.experimental.pallas import tpu as pltpu

_BN_EPS = 1e-5
_TB = 512
_RG = 4


def _round_up(x, m):
    return ((x + m - 1) // m) * m


def _nfm_kernel(featflat_ref, fvflat_ref,
                table_ref,
                w1_ref, b1_ref,
                w2_ref, b2_ref,
                wp_ref, bp_ref,
                out_ref,
                feat_smem, fv_smem, fm_ref, sq_ref, sem):
    num_fields = featflat_ref.shape[2] // _TB
    k_dim = table_ref.shape[2]

    cp_i = pltpu.make_async_copy(featflat_ref.at[0, 0], feat_smem, sem)
    cp_i.start()
    cp_i.wait()
    cp_v = pltpu.make_async_copy(fvflat_ref.at[0, 0], fv_smem, sem)
    cp_v.start()
    cp_v.wait()

    def group_body(i, carry):
        b0 = pl.multiple_of(i * _RG, _RG)
        s_rows = []
        sq_rows = []
        for r in range(_RG):
            base = (b0 + r) * num_fields
            s = jnp.zeros((1, k_dim), jnp.float32)
            sq = jnp.zeros((1, k_dim), jnp.float32)
            for f in range(num_fields):
                row = table_ref[feat_smem[base + f]]
                ne = row * fv_smem[base + f]
                s = s + ne
                sq = sq + ne * ne
            s_rows.append(s)
            sq_rows.append(sq)
        fm_ref[pl.ds(b0, _RG), :] = jnp.concatenate(s_rows, axis=0)
        sq_ref[pl.ds(b0, _RG), :] = jnp.concatenate(sq_rows, axis=0)
        return carry

    tb = out_ref.shape[0]
    jax.lax.fori_loop(0, tb // _RG, group_body, 0)

    sT = fm_ref[...]
    raw = sT * sT - sq_ref[...]
    h = jnp.dot(raw, w1_ref[...], preferred_element_type=jnp.float32)
    h = jnp.maximum(h + b1_ref[...], 0.0)
    h = jnp.dot(h, w2_ref[...], preferred_element_type=jnp.float32)
    h = jnp.maximum(h + b2_ref[...], 0.0)
    out_ref[...] = (jnp.sum(h * wp_ref[...], axis=1, keepdims=True)
                    + bp_ref[...])


def kernel(features, feature_values, embeddings, g0, b0, m0, v0,
           w1, bb1, g1, be1, m1, v1, w2, bb2, g2, be2, m2, v2, wp, bp):
    B, F = features.shape
    V, K = embeddings.shape
    H1 = w1.shape[1]
    H2 = w2.shape[1]

    inv0 = jax.lax.rsqrt(v0 + _BN_EPS)
    s0 = g0 * inv0
    t0 = b0 - m0 * s0

    inv1 = jax.lax.rsqrt(v1 + _BN_EPS)
    sc1 = g1 * inv1
    w1f = w1 * sc1
    b1f = (bb1 - m1) * sc1 + be1

    inv2 = jax.lax.rsqrt(v2 + _BN_EPS)
    sc2 = g2 * inv2
    w2f = w2 * sc2
    b2f = (bb2 - m2) * sc2 + be2

    wA = (0.5 * s0).reshape(K, 1) * w1f
    bA = t0 @ w1f + b1f

    table3 = embeddings.astype(jnp.float32).reshape(V, 1, K)

    Bp = _round_up(B, _TB)
    feat = features.astype(jnp.int32)
    fv = feature_values.astype(jnp.float32)
    if Bp != B:
        feat = jnp.pad(feat, ((0, Bp - B), (0, 0)))
        fv = jnp.pad(fv, ((0, Bp - B), (0, 0)))
    nsteps = Bp // _TB
    featflat = feat.reshape(nsteps, 1, _TB * F)
    fvflat = fv.reshape(nsteps, 1, _TB * F)

    const2d = lambda i: (0, 0)
    const3d = lambda i: (0, 0, 0)
    weight_args = (wA, bA, w2f, b2f, wp.reshape(1, H2), bp.reshape(1, 1))
    weight_specs = [
        pl.BlockSpec((K, H1), const2d), pl.BlockSpec((1, H1), const2d),
        pl.BlockSpec((H1, H2), const2d), pl.BlockSpec((1, H2), const2d),
        pl.BlockSpec((1, H2), const2d), pl.BlockSpec((1, 1), const2d),
    ]

    flops = Bp * (6 * F * K + 2 * K * H1 + 2 * H1 * H2 + 2 * H2 + 2 * K)
    bytes_accessed = 4 * (2 * Bp * F + Bp + V * K
                          + K * H1 + H1 * H2 + H1 + 2 * H2 + 1)

    out = pl.pallas_call(
        _nfm_kernel,
        out_shape=jax.ShapeDtypeStruct((Bp, 1), jnp.float32),
        grid=(nsteps,),
        in_specs=[
            pl.BlockSpec((1, 1, _TB * F), lambda i: (i, 0, 0)),
            pl.BlockSpec((1, 1, _TB * F), lambda i: (i, 0, 0)),
            pl.BlockSpec((V, 1, K), const3d),
        ] + weight_specs,
        out_specs=pl.BlockSpec((_TB, 1), lambda i: (i, 0)),
        scratch_shapes=[
            pltpu.SMEM((_TB * F,), jnp.int32),
            pltpu.SMEM((_TB * F,), jnp.float32),
            pltpu.VMEM((_TB, K), jnp.float32),
            pltpu.VMEM((_TB, K), jnp.float32),
            pltpu.SemaphoreType.DMA,
        ],
        compiler_params=pltpu.CompilerParams(
            dimension_semantics=("parallel",),
            vmem_limit_bytes=64 * 1024 * 1024,
        ),
        cost_estimate=pl.CostEstimate(
            flops=int(flops), transcendentals=0,
            bytes_accessed=int(bytes_accessed)),
    )(featflat, fvflat, table3, *weight_args)
    return out[:B, 0]


# speedup vs baseline: 1.1811x; 1.1811x over previous
import jax
import jax.numpy as jnp
from jax.experimental import pallas as pl
from jax.experimental.pallas import tpu as pltpu

_BN_EPS = 1e-5
_TB = 1024
_RG = 2


def _round_up(x, m):
    return ((x + m - 1) // m) * m


def _nfm_kernel(featflat_ref, fvflat_ref,
                table_ref,
                w1_ref, b1_ref,
                w2_ref, b2_ref,
                wp_ref, bp_ref,
                out_ref,
                feat_smem, fv_smem, fm_ref, sq_ref, sem):
    num_fields = featflat_ref.shape[2] // _TB
    k_dim = table_ref.shape[2]

    cp_i = pltpu.make_async_copy(featflat_ref.at[0, 0], feat_smem, sem)
    cp_i.start()
    cp_i.wait()
    cp_v = pltpu.make_async_copy(fvflat_ref.at[0, 0], fv_smem, sem)
    cp_v.start()
    cp_v.wait()

    def group_body(i, carry):
        b0 = pl.multiple_of(i * _RG, _RG)
        s_rows = []
        sq_rows = []
        for r in range(_RG):
            base = (b0 + r) * num_fields
            s = jnp.zeros((1, k_dim), jnp.float32)
            sq = jnp.zeros((1, k_dim), jnp.float32)
            for f in range(num_fields):
                row = table_ref[feat_smem[base + f]]
                ne = row * fv_smem[base + f]
                s = s + ne
                sq = sq + ne * ne
            s_rows.append(s)
            sq_rows.append(sq)
        fm_ref[pl.ds(b0, _RG), :] = jnp.concatenate(s_rows, axis=0)
        sq_ref[pl.ds(b0, _RG), :] = jnp.concatenate(sq_rows, axis=0)
        return carry

    tb = out_ref.shape[0]
    jax.lax.fori_loop(0, tb // _RG, group_body, 0)

    sT = fm_ref[...]
    raw = sT * sT - sq_ref[...]
    h = jnp.dot(raw, w1_ref[...], preferred_element_type=jnp.float32)
    h = jnp.maximum(h + b1_ref[...], 0.0)
    h = jnp.dot(h, w2_ref[...], preferred_element_type=jnp.float32)
    h = jnp.maximum(h + b2_ref[...], 0.0)
    out_ref[...] = (jnp.sum(h * wp_ref[...], axis=1, keepdims=True)
                    + bp_ref[...])


def kernel(features, feature_values, embeddings, g0, b0, m0, v0,
           w1, bb1, g1, be1, m1, v1, w2, bb2, g2, be2, m2, v2, wp, bp):
    B, F = features.shape
    V, K = embeddings.shape
    H1 = w1.shape[1]
    H2 = w2.shape[1]

    inv0 = jax.lax.rsqrt(v0 + _BN_EPS)
    s0 = g0 * inv0
    t0 = b0 - m0 * s0

    inv1 = jax.lax.rsqrt(v1 + _BN_EPS)
    sc1 = g1 * inv1
    w1f = w1 * sc1
    b1f = (bb1 - m1) * sc1 + be1

    inv2 = jax.lax.rsqrt(v2 + _BN_EPS)
    sc2 = g2 * inv2
    w2f = w2 * sc2
    b2f = (bb2 - m2) * sc2 + be2

    wA = (0.5 * s0).reshape(K, 1) * w1f
    bA = t0 @ w1f + b1f

    table3 = embeddings.astype(jnp.float32).reshape(V, 1, K)

    Bp = _round_up(B, _TB)
    feat = features.astype(jnp.int32)
    fv = feature_values.astype(jnp.float32)
    if Bp != B:
        feat = jnp.pad(feat, ((0, Bp - B), (0, 0)))
        fv = jnp.pad(fv, ((0, Bp - B), (0, 0)))
    nsteps = Bp // _TB
    featflat = feat.reshape(nsteps, 1, _TB * F)
    fvflat = fv.reshape(nsteps, 1, _TB * F)

    const2d = lambda i: (0, 0)
    const3d = lambda i: (0, 0, 0)
    weight_args = (wA, bA, w2f, b2f, wp.reshape(1, H2), bp.reshape(1, 1))
    weight_specs = [
        pl.BlockSpec((K, H1), const2d), pl.BlockSpec((1, H1), const2d),
        pl.BlockSpec((H1, H2), const2d), pl.BlockSpec((1, H2), const2d),
        pl.BlockSpec((1, H2), const2d), pl.BlockSpec((1, 1), const2d),
    ]

    flops = Bp * (6 * F * K + 2 * K * H1 + 2 * H1 * H2 + 2 * H2 + 2 * K)
    bytes_accessed = 4 * (2 * Bp * F + Bp + V * K
                          + K * H1 + H1 * H2 + H1 + 2 * H2 + 1)

    out = pl.pallas_call(
        _nfm_kernel,
        out_shape=jax.ShapeDtypeStruct((Bp, 1), jnp.float32),
        grid=(nsteps,),
        in_specs=[
            pl.BlockSpec((1, 1, _TB * F), lambda i: (i, 0, 0)),
            pl.BlockSpec((1, 1, _TB * F), lambda i: (i, 0, 0)),
            pl.BlockSpec((V, 1, K), const3d),
        ] + weight_specs,
        out_specs=pl.BlockSpec((_TB, 1), lambda i: (i, 0)),
        scratch_shapes=[
            pltpu.SMEM((_TB * F,), jnp.int32),
            pltpu.SMEM((_TB * F,), jnp.float32),
            pltpu.VMEM((_TB, K), jnp.float32),
            pltpu.VMEM((_TB, K), jnp.float32),
            pltpu.SemaphoreType.DMA,
        ],
        compiler_params=pltpu.CompilerParams(
            dimension_semantics=("parallel",),
            vmem_limit_bytes=64 * 1024 * 1024,
        ),
        cost_estimate=pl.CostEstimate(
            flops=int(flops), transcendentals=0,
            bytes_accessed=int(bytes_accessed)),
    )(featflat, fvflat, table3, *weight_args)
    return out[:B, 0]
